# baseline (speedup 1.0000x reference)
import functools

import jax
import jax.numpy as jnp
from jax import lax
from jax.experimental import pallas as pl
from jax.experimental.pallas import tpu as pltpu
from jax.experimental.pallas import tpu_sc as plsc

_NUM_WORKERS = 32
_CHUNK = 640


def kernel(inp_data, emb_table):
    B, T = inp_data.shape
    V, E = emb_table.shape
    N = B * T
    CB = (T * E) // 128
    RB = B // 8
    NZ = N // 2
    z_per_w = NZ // _NUM_WORKERS
    n_chunks = z_per_w // _CHUNK

    idx4 = inp_data.astype(jnp.int32).reshape(RB, 8, CB, 2)
    idx4 = idx4.transpose(0, 2, 1, 3)
    idx_even = idx4[..., 0].reshape(NZ)
    idx_odd = idx4[..., 1].reshape(NZ)

    mesh = plsc.VectorSubcoreMesh(core_axis_name="c", subcore_axis_name="s")

    @functools.partial(
        pl.kernel,
        mesh=mesh,
        compiler_params=pltpu.CompilerParams(use_tc_tiling_on_sc=False),
        out_type=jax.ShapeDtypeStruct((NZ, 2 * E), emb_table.dtype),
        scratch_types=[
            pltpu.VMEM((_CHUNK,), jnp.int32),
            pltpu.VMEM((_CHUNK,), jnp.int32),
            pltpu.VMEM((_CHUNK, E), emb_table.dtype),
            pltpu.VMEM((_CHUNK, E), emb_table.dtype),
            pltpu.SemaphoreType.DMA,
        ],
    )
    def gather_kernel(tbl_hbm, ie_hbm, io_hbm, z_hbm,
                      ie_v, io_v, re_v, ro_v, sem):
        wid = lax.axis_index("s") * 2 + lax.axis_index("c")
        base = wid * z_per_w

        @pl.loop(0, n_chunks)
        def _(c):
            off = base + c * _CHUNK
            pltpu.sync_copy(ie_hbm.at[pl.ds(off, _CHUNK)], ie_v)
            pltpu.sync_copy(io_hbm.at[pl.ds(off, _CHUNK)], io_v)
            ce = pltpu.async_copy(tbl_hbm.at[ie_v], re_v, sem)
            co = pltpu.async_copy(tbl_hbm.at[io_v], ro_v, sem)
            ce.wait()
            co.wait()
            pltpu.sync_copy(re_v, z_hbm.at[pl.ds(off, _CHUNK), pl.ds(0, E)])
            pltpu.sync_copy(ro_v, z_hbm.at[pl.ds(off, _CHUNK), pl.ds(E, E)])

    z = gather_kernel(emb_table, idx_even, idx_odd)
    out = z.reshape(RB, CB, 8, 2 * E).transpose(0, 2, 1, 3)
    return out.reshape(B, T, E)


# speedup vs baseline: 1.7761x; 1.7761x over previous
import functools

import jax
import jax.numpy as jnp
import numpy as np
from jax import lax
from jax.experimental import pallas as pl
from jax.experimental.pallas import tpu as pltpu
from jax.experimental.pallas import tpu_sc as plsc

_NUM_WORKERS = 32
_ZCHUNK = 640
_GB = 32


def kernel(inp_data, emb_table):
    B, T = inp_data.shape
    V, E = emb_table.shape
    N = B * T
    CB = (T * E) // 128
    RB = B // 8
    NZ = N // 2
    z_per_w = NZ // _NUM_WORKERS
    n_chunks = z_per_w // _ZCHUNK
    i_chunk = 2 * _ZCHUNK
    tb_per_chunk = i_chunk // (8 * T)

    jz = np.arange(_ZCHUNK)
    r = jz % 8
    t = (jz // 8) % CB
    tb = jz // (8 * CB)
    pos_pat = jnp.asarray(8 * T * tb + T * r + 2 * t, dtype=jnp.int32)

    idx = inp_data.reshape(N).astype(jnp.int32)

    mesh = plsc.VectorSubcoreMesh(core_axis_name="c", subcore_axis_name="s")

    @functools.partial(
        pl.kernel,
        mesh=mesh,
        compiler_params=pltpu.CompilerParams(use_tc_tiling_on_sc=False,
                                             needs_layout_passes=False),
        out_type=jax.ShapeDtypeStruct((NZ, 2 * E), emb_table.dtype),
        scratch_types=[
            pltpu.VMEM((_ZCHUNK,), jnp.int32),
            pltpu.VMEM((i_chunk,), jnp.int32),
            pltpu.VMEM((_ZCHUNK,), jnp.int32),
            pltpu.VMEM((_ZCHUNK,), jnp.int32),
            pltpu.VMEM((_ZCHUNK, E), emb_table.dtype),
            pltpu.VMEM((_ZCHUNK, E), emb_table.dtype),
            pltpu.SemaphoreType.DMA,
        ],
    )
    def gather_kernel(tbl_hbm, idx_hbm, pos_hbm, z_hbm,
                      pos_v, idx_v, ie_v, io_v, re_v, ro_v, sem):
        wid = lax.axis_index("s") * 2 + lax.axis_index("c")
        zbase = wid * z_per_w
        pltpu.sync_copy(pos_hbm, pos_v)

        @pl.loop(0, n_chunks)
        def _(c):
            zoff = zbase + c * _ZCHUNK
            pltpu.sync_copy(idx_hbm.at[pl.ds(2 * zoff, i_chunk)], idx_v)
            for v in range(_ZCHUNK // 16):
                p = pos_v[pl.ds(16 * v, 16)]
                ie_v[pl.ds(16 * v, 16)] = plsc.load_gather(idx_v, [p])
                io_v[pl.ds(16 * v, 16)] = plsc.load_gather(idx_v, [p + 1])
            ce = pltpu.async_copy(tbl_hbm.at[ie_v], re_v, sem)
            co = pltpu.async_copy(tbl_hbm.at[io_v], ro_v, sem)
            ce.wait()
            co.wait()
            pltpu.sync_copy(re_v, z_hbm.at[pl.ds(zoff, _ZCHUNK), pl.ds(0, E)])
            pltpu.sync_copy(ro_v, z_hbm.at[pl.ds(zoff, _ZCHUNK), pl.ds(E, E)])

    z = gather_kernel(emb_table, idx, pos_pat)

    z4 = z.reshape(RB, CB, 8, 2 * E)

    def untile_body(x_ref, o_ref):
        for tc in range(CB):
            o_ref[:, 128 * tc:128 * (tc + 1)] = (
                x_ref[:, tc].reshape(_GB * 8, 128))

    y = pl.pallas_call(
        untile_body,
        grid=(RB // _GB,),
        in_specs=[pl.BlockSpec((_GB, CB, 8, 2 * E), lambda i: (i, 0, 0, 0))],
        out_specs=pl.BlockSpec((_GB * 8, T * E), lambda i: (i, 0)),
        out_shape=jax.ShapeDtypeStruct((B, T * E), emb_table.dtype),
    )(z4)
    return y.reshape(B, T, E)
